# Initial kernel scaffold; baseline (speedup 1.0000x reference)
import jax
import jax.numpy as jnp
from jax.experimental import pallas as pl


def kernel(feature_p3, feature_p4, feature_p5, feature_p6, boxes, nms_classification):
    raise NotImplementedError("write your pallas kernel here")


# speedup vs baseline: 20.0366x; 20.0366x over previous
import jax
import jax.numpy as jnp
from jax.experimental import pallas as pl
from jax.experimental.pallas import tpu as pltpu

_POOL = 7
_K2 = 20
_NP = 10
_THR = 0.05
_NC = 80
_RL = 1280
_NR = 1250
_NRP = 1256
_NRB = _NRP // 8


def _select_kernel(s_ref, boxes_ref, obox_ref, oval_ref, rowmax_ref):
    def body(i, carry):
        blk = s_ref[pl.ds(i * 8, 8), :]
        m = jnp.where(blk > _THR, blk, -jnp.inf)
        rmax = jnp.max(m, axis=1)
        rowmax_ref[pl.ds(i, 1), :] = rmax.reshape(1, 8)
        return carry

    jax.lax.fori_loop(0, _NRB, body, 0)

    rm = rowmax_ref[:, :]
    riota = (jax.lax.broadcasted_iota(jnp.int32, (_NRB, 8), 0) * 8
             + jax.lax.broadcasted_iota(jnp.int32, (_NRB, 8), 1))
    ciota = jax.lax.broadcasted_iota(jnp.int32, (1, _RL), 1)
    big = jnp.int32(1 << 30)
    neg_inf = jnp.float32(-jnp.inf)

    extracted = []
    for _ in range(_K2):
        m = jnp.max(rm)
        rstar = jnp.min(jnp.where(rm == m, riota, big))
        row = s_ref[pl.ds(rstar, 1), :]
        rowm = jnp.where(row > _THR, row, neg_inf)
        for (rj, cj) in extracted:
            rowm = jnp.where((ciota == cj) & (rstar == rj), neg_inf, rowm)
        cstar = jnp.min(jnp.where(rowm == m, ciota, big))
        extracted.append((rstar, cstar))
        rowm2 = jnp.where(ciota == cstar, neg_inf, rowm)
        rm = jnp.where(riota == rstar, jnp.max(rowm2), rm)

    bs = [(r * _RL + c) // _NC for (r, c) in extracted]
    firsts = []
    for j in range(_K2):
        fj = None
        for k in range(j):
            ne = bs[j] != bs[k]
            fj = ne if fj is None else (fj & ne)
        firsts.append(jnp.bool_(True) if fj is None else fj)
    ranks = []
    acc = jnp.int32(0)
    for j in range(_K2):
        ranks.append(acc)
        acc = acc + firsts[j].astype(jnp.int32)

    for s in range(_NP):
        sel = jnp.int32(0)
        val = jnp.float32(0.0)
        for j in range(_K2):
            hit = firsts[j] & (ranks[j] == s)
            sel = jnp.where(hit, bs[j], sel)
            val = jnp.where(hit, jnp.float32(1.0), val)
        rowb = boxes_ref[pl.ds(sel, 1), :]
        obox_ref[pl.ds(s, 1), 0:4] = rowb
        oval_ref[pl.ds(s, 1), :] = jnp.full((1, 128), val, jnp.float32)


def _axis_weights(c1, c2, size, valid):
    raise NotImplementedError


def _interp_matrix(c1, c2, size, cell_idx):
    pos = jax.lax.broadcasted_iota(jnp.int32, (56, size), 1)
    t = cell_idx.astype(jnp.float32) / jnp.float32(_POOL - 1)
    a = c1 * jnp.float32(size - 1)
    span = (c2 - c1) * jnp.float32(size - 1)
    in_c = a + t * span
    c0f = jnp.floor(in_c)
    w = in_c - c0f
    c0 = jnp.clip(c0f, 0, size - 1).astype(jnp.int32)
    c1i = jnp.clip(c0f + 1.0, 0, size - 1).astype(jnp.int32)
    return ((1.0 - w) * (pos == c0).astype(jnp.float32)
            + w * (pos == c1i).astype(jnp.float32))


def _crop_kernel(f3_ref, f4_ref, f5_ref, f6_ref, obox_ref, oval_ref, out_ref):
    b = pl.program_id(0)
    x1 = obox_ref[b, 0]
    y1 = obox_ref[b, 1]
    x2 = obox_ref[b, 2]
    y2 = obox_ref[b, 3]
    valid = oval_ref[b, 0]

    acc = None
    for f_ref, size in ((f3_ref, 64), (f4_ref, 32), (f5_ref, 16), (f6_ref, 8)):
        r0 = jax.lax.broadcasted_iota(jnp.int32, (56, size), 0)
        iy = r0 // _POOL
        ix = r0 % _POOL
        wy = _interp_matrix(y1, y2, size, iy) * valid
        wx = _interp_matrix(x1, x2, size, ix)
        k = (wy[:, :, None] * wx[:, None, :]).reshape(56, size * size)
        lvl = jnp.dot(k, f_ref[:, :])
        acc = lvl if acc is None else jnp.maximum(acc, lvl)
    out_ref[0, :, :] = acc[0:_POOL * _POOL, :]


def kernel(feature_p3, feature_p4, feature_p5, feature_p6, boxes, nms_classification):
    s2 = nms_classification[0].reshape(_NR, _RL)
    s2 = jnp.pad(s2, ((0, _NRP - _NR), (0, 0)))
    bx = boxes[0]

    obox, oval = pl.pallas_call(
        _select_kernel,
        out_shape=[
            jax.ShapeDtypeStruct((16, 128), jnp.float32),
            jax.ShapeDtypeStruct((16, 128), jnp.float32),
        ],
        scratch_shapes=[pltpu.VMEM((_NRB, 8), jnp.float32)],
    )(s2, bx)

    fr = [f[0].reshape(-1, 256) for f in
          (feature_p3, feature_p4, feature_p5, feature_p6)]

    out = pl.pallas_call(
        _crop_kernel,
        grid=(_NP,),
        in_specs=[
            pl.BlockSpec((64 * 64, 256), lambda b: (0, 0)),
            pl.BlockSpec((32 * 32, 256), lambda b: (0, 0)),
            pl.BlockSpec((16 * 16, 256), lambda b: (0, 0)),
            pl.BlockSpec((8 * 8, 256), lambda b: (0, 0)),
            pl.BlockSpec(memory_space=pltpu.SMEM),
            pl.BlockSpec(memory_space=pltpu.SMEM),
        ],
        out_specs=pl.BlockSpec((1, _POOL * _POOL, 256), lambda b: (b, 0, 0)),
        out_shape=jax.ShapeDtypeStruct((_NP, _POOL * _POOL, 256), jnp.float32),
    )(fr[0], fr[1], fr[2], fr[3], obox, oval)

    return out.reshape(1, _NP, _POOL, _POOL, 256)


# speedup vs baseline: 24.7087x; 1.2332x over previous
import jax
import jax.numpy as jnp
from jax.experimental import pallas as pl
from jax.experimental.pallas import tpu as pltpu

_POOL = 7
_K2 = 20
_NP = 10
_THR = 0.05
_NC = 80
_NCH = 2500
_NSC = 320


def _select_kernel(s_ref, boxes_ref, obox_ref, oval_ref, cm3_ref):
    neg_inf = jnp.float32(-jnp.inf)
    big = jnp.int32(1 << 30)

    def body(i, carry):
        blk = s_ref[pl.ds(i * 64, 64), :, :]
        cmr = jnp.max(blk, axis=1)
        cmr = jnp.where(cmr > _THR, cmr, neg_inf)
        cm3_ref[pl.ds(i * 8, 8), :, :] = cmr.reshape(8, 8, 80)
        return carry

    jax.lax.fori_loop(0, 39, body, 0)
    tail = jnp.max(s_ref[pl.ds(2496, 4), :, :], axis=1)
    tail = jnp.where(tail > _THR, tail, neg_inf)
    tail = jnp.concatenate([tail, jnp.full((60, 80), neg_inf)], axis=0)
    cm3_ref[pl.ds(312, 8), :, :] = tail.reshape(8, 8, 80)

    cm2 = jnp.max(cm3_ref[:, :, :], axis=1)

    i0_sc = jax.lax.broadcasted_iota(jnp.int32, (_NSC, 80), 0)
    i0_8 = jax.lax.broadcasted_iota(jnp.int32, (8, 80), 0)
    i1_8 = jax.lax.broadcasted_iota(jnp.int32, (8, 80), 1)
    fid8 = i0_8 * _NC + i1_8

    extracted = []
    bs = []
    for _ in range(_K2):
        m = jnp.max(cm2)
        i2 = jnp.min(jnp.where(cm2 == m, i0_sc, big))
        crows = cm3_ref[pl.ds(i2, 1), :, :].reshape(8, 80)
        j2 = jnp.min(jnp.where(crows == m, i0_8, big))
        chunk = i2 * 8 + j2
        blk = s_ref[pl.ds(chunk, 1), :, :].reshape(8, 80)
        blkm = jnp.where(blk > _THR, blk, neg_inf)
        for (ck, jk, cck) in extracted:
            blkm = jnp.where((chunk == ck) & (i0_8 == jk) & (i1_8 == cck),
                             neg_inf, blkm)
        v = jnp.min(jnp.where(blkm == m, fid8, big))
        j3 = v // _NC
        c = v - j3 * _NC
        extracted.append((chunk, j3, c))
        bs.append(chunk * 8 + j3)
        blkm2 = jnp.where((i0_8 == j3) & (i1_8 == c), neg_inf, blkm)
        newrow = jnp.max(blkm2, axis=0, keepdims=True)
        newslice = jnp.where(i0_8 == j2, newrow, crows)
        cm3_ref[pl.ds(i2, 1), :, :] = newslice.reshape(1, 8, 80)
        new2 = jnp.max(newslice, axis=0, keepdims=True)
        cm2 = jnp.where(i0_sc == i2, new2, cm2)

    firsts = []
    for j in range(_K2):
        fj = None
        for k in range(j):
            ne = bs[j] != bs[k]
            fj = ne if fj is None else (fj & ne)
        firsts.append(jnp.bool_(True) if fj is None else fj)
    ranks = []
    acc = jnp.int32(0)
    for j in range(_K2):
        ranks.append(acc)
        acc = acc + firsts[j].astype(jnp.int32)

    for s in range(_NP):
        sel = jnp.int32(0)
        val = jnp.float32(0.0)
        for j in range(_K2):
            hit = firsts[j] & (ranks[j] == s)
            sel = jnp.where(hit, bs[j], sel)
            val = jnp.where(hit, jnp.float32(1.0), val)
        rowb = boxes_ref[pl.ds(sel, 1), :]
        obox_ref[pl.ds(s, 1), 0:4] = rowb
        oval_ref[pl.ds(s, 1), :] = jnp.full((1, 128), val, jnp.float32)


def _interp_matrix(c1, c2, size, cell_idx):
    pos = jax.lax.broadcasted_iota(jnp.int32, (56, size), 1)
    t = cell_idx.astype(jnp.float32) / jnp.float32(_POOL - 1)
    a = c1 * jnp.float32(size - 1)
    span = (c2 - c1) * jnp.float32(size - 1)
    in_c = a + t * span
    c0f = jnp.floor(in_c)
    w = in_c - c0f
    c0 = jnp.clip(c0f, 0, size - 1).astype(jnp.int32)
    c1i = jnp.clip(c0f + 1.0, 0, size - 1).astype(jnp.int32)
    return ((1.0 - w) * (pos == c0).astype(jnp.float32)
            + w * (pos == c1i).astype(jnp.float32))


def _crop_kernel(f3_ref, f4_ref, f5_ref, f6_ref, obox_ref, oval_ref, out_ref):
    b = pl.program_id(0)
    x1 = obox_ref[b, 0]
    y1 = obox_ref[b, 1]
    x2 = obox_ref[b, 2]
    y2 = obox_ref[b, 3]
    valid = oval_ref[b, 0]

    acc = None
    for f_ref, size in ((f3_ref, 64), (f4_ref, 32), (f5_ref, 16), (f6_ref, 8)):
        r0 = jax.lax.broadcasted_iota(jnp.int32, (56, size), 0)
        iy = r0 // _POOL
        ix = r0 % _POOL
        wy = _interp_matrix(y1, y2, size, iy) * valid
        wx = _interp_matrix(x1, x2, size, ix)
        k = (wy[:, :, None] * wx[:, None, :]).reshape(56, size * size)
        lvl = jnp.dot(k, f_ref[:, :])
        acc = lvl if acc is None else jnp.maximum(acc, lvl)
    out_ref[0, :, :] = acc[0:_POOL * _POOL, :]


def kernel(feature_p3, feature_p4, feature_p5, feature_p6, boxes, nms_classification):
    s3 = nms_classification.reshape(_NCH, 8, _NC)
    bx = boxes[0]

    obox, oval = pl.pallas_call(
        _select_kernel,
        out_shape=[
            jax.ShapeDtypeStruct((16, 128), jnp.float32),
            jax.ShapeDtypeStruct((16, 128), jnp.float32),
        ],
        scratch_shapes=[pltpu.VMEM((_NSC, 8, _NC), jnp.float32)],
    )(s3, bx)

    fr = [f[0].reshape(-1, 256) for f in
          (feature_p3, feature_p4, feature_p5, feature_p6)]

    out = pl.pallas_call(
        _crop_kernel,
        grid=(_NP,),
        in_specs=[
            pl.BlockSpec((64 * 64, 256), lambda b: (0, 0)),
            pl.BlockSpec((32 * 32, 256), lambda b: (0, 0)),
            pl.BlockSpec((16 * 16, 256), lambda b: (0, 0)),
            pl.BlockSpec((8 * 8, 256), lambda b: (0, 0)),
            pl.BlockSpec(memory_space=pltpu.SMEM),
            pl.BlockSpec(memory_space=pltpu.SMEM),
        ],
        out_specs=pl.BlockSpec((1, _POOL * _POOL, 256), lambda b: (b, 0, 0)),
        out_shape=jax.ShapeDtypeStruct((_NP, _POOL * _POOL, 256), jnp.float32),
    )(fr[0], fr[1], fr[2], fr[3], obox, oval)

    return out.reshape(1, _NP, _POOL, _POOL, 256)


# speedup vs baseline: 52.0018x; 2.1046x over previous
import jax
import jax.numpy as jnp
from jax.experimental import pallas as pl
from jax.experimental.pallas import tpu as pltpu

_POOL = 7
_K2 = 20
_NP = 10
_THR = 0.05
_NC = 80
_NB = 20000
_NCH = 156
_TAIL = _NB - _NCH * 128
_NROW = 160


def _select(s_ref, bx_ref, cm1_ref, sel_ref):
    neg_inf = jnp.float32(-jnp.inf)
    big = jnp.int32(1 << 30)

    i0c = jax.lax.broadcasted_iota(jnp.int32, (_NC, 128), 0)
    i1c = jax.lax.broadcasted_iota(jnp.int32, (_NC, 128), 1)

    def body(i, carry):
        blk = s_ref[:, pl.ds(pl.multiple_of(i * 128, 128), 128)]
        blkm = jnp.where(blk > _THR, blk, neg_inf)
        cm1_ref[pl.ds(i, 1), :] = jnp.max(blkm, axis=0, keepdims=True)
        return carry

    jax.lax.fori_loop(0, 152, body, 0)
    cm1_ref[pl.ds(152, 8), :] = jnp.full((8, 128), neg_inf, jnp.float32)
    tb = s_ref[:, pl.ds(_NCH * 128, _TAIL)]
    tbm = jnp.where(tb > _THR, tb, neg_inf)
    tmax = jnp.max(tbm, axis=0, keepdims=True)
    cm1_ref[pl.ds(_NCH, 1), :] = jnp.concatenate(
        [tmax, jnp.full((1, 128 - _TAIL), neg_inf, jnp.float32)], axis=1)
    for i in range(152, _NCH):
        blk = s_ref[:, i * 128:(i + 1) * 128]
        blkm = jnp.where(blk > _THR, blk, neg_inf)
        cm1_ref[pl.ds(i, 1), :] = jnp.max(blkm, axis=0, keepdims=True)

    cm1 = cm1_ref[:, :]
    i0r = jax.lax.broadcasted_iota(jnp.int32, (_NROW, 128), 0)
    i1r = jax.lax.broadcasted_iota(jnp.int32, (_NROW, 128), 1)

    extracted = []
    bs = []
    for _ in range(_K2):
        m = jnp.max(cm1)
        chunk = jnp.min(jnp.where(cm1 == m, i0r, big))
        rowv = jnp.max(jnp.where(i0r == chunk, cm1, neg_inf), axis=0,
                       keepdims=True)
        lane = jnp.min(jnp.where(rowv == m,
                                 jax.lax.broadcasted_iota(jnp.int32, (1, 128), 1),
                                 big))
        b = chunk * 128 + lane
        sub = s_ref[:, pl.ds(pl.multiple_of(chunk * 128, 128), 128)]
        subm = jnp.where((sub > _THR) & (chunk * 128 + i1c < _NB), sub,
                         neg_inf)
        for (bk, ck) in extracted:
            subm = jnp.where((chunk * 128 + i1c == bk) & (i0c == ck),
                             neg_inf, subm)
        c = jnp.min(jnp.where((i1c == lane) & (subm == m), i0c, big))
        extracted.append((b, c))
        bs.append(b)
        subm2 = jnp.where((i1c == lane) & (i0c == c), neg_inf, subm)
        newrow = jnp.max(subm2, axis=0, keepdims=True)
        cm1 = jnp.where(i0r == chunk, newrow, cm1)

    firsts = []
    for j in range(_K2):
        fj = None
        for k in range(j):
            ne = bs[j] != bs[k]
            fj = ne if fj is None else (fj & ne)
        firsts.append(jnp.bool_(True) if fj is None else fj)
    ranks = []
    acc = jnp.int32(0)
    for j in range(_K2):
        ranks.append(acc)
        acc = acc + firsts[j].astype(jnp.int32)

    i1b = jax.lax.broadcasted_iota(jnp.int32, (4, 128), 1)
    for s in range(_NP):
        sel = jnp.int32(0)
        val = jnp.float32(0.0)
        for j in range(_K2):
            hit = firsts[j] & (ranks[j] == s)
            sel = jnp.where(hit, bs[j], sel)
            val = jnp.where(hit, jnp.float32(1.0), val)
        sc = sel // 128
        sl = sel - sc * 128
        bchunk = bx_ref[:, pl.ds(pl.multiple_of(sc * 128, 128), 128)]
        picked = jnp.where(i1b == sl, bchunk, neg_inf)
        for k in range(4):
            ck = jnp.max(picked[k:k + 1, :])
            sel_ref[pl.ds(s, 1), k:k + 1] = jnp.full((1, 1), ck, jnp.float32)
        sel_ref[pl.ds(s, 1), 4:5] = jnp.full((1, 1), val, jnp.float32)


def _interp_matrix(c1, c2, size, cell_idx):
    pos = jax.lax.broadcasted_iota(jnp.int32, (56, size), 1)
    t = cell_idx.astype(jnp.float32) / jnp.float32(_POOL - 1)
    a = c1 * jnp.float32(size - 1)
    span = (c2 - c1) * jnp.float32(size - 1)
    in_c = a + t * span
    c0f = jnp.floor(in_c)
    w = in_c - c0f
    c0 = jnp.clip(c0f, 0, size - 1).astype(jnp.int32)
    c1i = jnp.clip(c0f + 1.0, 0, size - 1).astype(jnp.int32)
    return ((1.0 - w) * (pos == c0).astype(jnp.float32)
            + w * (pos == c1i).astype(jnp.float32))


def _fused_kernel(s_ref, bx_ref, f3_ref, f4_ref, f5_ref, f6_ref, out_ref,
                  cm1_ref, sel_ref):
    b = pl.program_id(0)

    @pl.when(b == 0)
    def _():
        _select(s_ref, bx_ref, cm1_ref, sel_ref)

    x1 = sel_ref[pl.ds(b, 1), 0:1]
    y1 = sel_ref[pl.ds(b, 1), 1:2]
    x2 = sel_ref[pl.ds(b, 1), 2:3]
    y2 = sel_ref[pl.ds(b, 1), 3:4]
    valid = sel_ref[pl.ds(b, 1), 4:5]

    acc = None
    for f_ref, size in ((f3_ref, 64), (f4_ref, 32), (f5_ref, 16), (f6_ref, 8)):
        r0 = jax.lax.broadcasted_iota(jnp.int32, (56, size), 0)
        iy = r0 // _POOL
        ix = r0 % _POOL
        wy = _interp_matrix(y1, y2, size, iy) * valid
        wx = _interp_matrix(x1, x2, size, ix)
        k = (wy[:, :, None] * wx[:, None, :]).reshape(56, size * size)
        lvl = jnp.dot(k, f_ref[:, :])
        acc = lvl if acc is None else jnp.maximum(acc, lvl)
    for i in range(_POOL):
        out_ref[0, 0, i, :, :] = acc[i * _POOL:(i + 1) * _POOL, :]


def kernel(feature_p3, feature_p4, feature_p5, feature_p6, boxes, nms_classification):
    s_t = nms_classification[0].T
    bx_t = boxes[0].T
    fr = [f[0].reshape(-1, 256) for f in
          (feature_p3, feature_p4, feature_p5, feature_p6)]

    out = pl.pallas_call(
        _fused_kernel,
        grid=(_NP,),
        in_specs=[
            pl.BlockSpec((_NC, _NB), lambda b: (0, 0)),
            pl.BlockSpec((4, _NB), lambda b: (0, 0)),
            pl.BlockSpec((64 * 64, 256), lambda b: (0, 0)),
            pl.BlockSpec((32 * 32, 256), lambda b: (0, 0)),
            pl.BlockSpec((16 * 16, 256), lambda b: (0, 0)),
            pl.BlockSpec((8 * 8, 256), lambda b: (0, 0)),
        ],
        out_specs=pl.BlockSpec((1, 1, _POOL, _POOL, 256),
                               lambda b: (0, b, 0, 0, 0)),
        out_shape=jax.ShapeDtypeStruct((1, _NP, _POOL, _POOL, 256),
                                       jnp.float32),
        scratch_shapes=[
            pltpu.VMEM((_NROW, 128), jnp.float32),
            pltpu.VMEM((16, 128), jnp.float32),
        ],
    )(s_t, bx_t, fr[0], fr[1], fr[2], fr[3])

    return out


# speedup vs baseline: 56.3485x; 1.0836x over previous
import jax
import jax.numpy as jnp
from jax.experimental import pallas as pl
from jax.experimental.pallas import tpu as pltpu

_POOL = 7
_K2 = 20
_NP = 10
_THR = 0.05
_NC = 80
_NB = 20000
_NCH = 156
_TAIL = _NB - _NCH * 128
_NROW = 160


def _select(s_ref, bx_ref, cm1_ref, sel_ref):
    neg_inf = jnp.float32(-jnp.inf)
    big = jnp.int32(1 << 30)

    i0c = jax.lax.broadcasted_iota(jnp.int32, (_NC, 128), 0)
    i1c = jax.lax.broadcasted_iota(jnp.int32, (_NC, 128), 1)

    def body(i, carry):
        blk = s_ref[:, pl.ds(pl.multiple_of(i * 128, 128), 128)]
        blkm = jnp.where(blk > _THR, blk, neg_inf)
        cm1_ref[pl.ds(i, 1), :] = jnp.max(blkm, axis=0, keepdims=True)
        return carry

    jax.lax.fori_loop(0, 152, body, 0)
    cm1_ref[pl.ds(152, 8), :] = jnp.full((8, 128), neg_inf, jnp.float32)
    tb = s_ref[:, pl.ds(_NCH * 128, _TAIL)]
    tbm = jnp.where(tb > _THR, tb, neg_inf)
    tmax = jnp.max(tbm, axis=0, keepdims=True)
    cm1_ref[pl.ds(_NCH, 1), :] = jnp.concatenate(
        [tmax, jnp.full((1, 128 - _TAIL), neg_inf, jnp.float32)], axis=1)
    for i in range(152, _NCH):
        blk = s_ref[:, i * 128:(i + 1) * 128]
        blkm = jnp.where(blk > _THR, blk, neg_inf)
        cm1_ref[pl.ds(i, 1), :] = jnp.max(blkm, axis=0, keepdims=True)

    cm1 = cm1_ref[:, :]
    i0r = jax.lax.broadcasted_iota(jnp.int32, (_NROW, 128), 0)
    i1r = jax.lax.broadcasted_iota(jnp.int32, (_NROW, 128), 1)
    boxid = i0r * 128 + i1r

    extracted = []
    bs = []
    for _ in range(_K2):
        m = jnp.max(cm1)
        b = jnp.min(jnp.where(cm1 == m, boxid, big))
        chunk = b // 128
        lane = b - chunk * 128
        sub = s_ref[:, pl.ds(pl.multiple_of(chunk * 128, 128), 128)]
        subm = jnp.where((sub > _THR) & (chunk * 128 + i1c < _NB), sub,
                         neg_inf)
        for (bk, ck) in extracted:
            subm = jnp.where((chunk * 128 + i1c == bk) & (i0c == ck),
                             neg_inf, subm)
        c = jnp.min(jnp.where((i1c == lane) & (subm == m), i0c, big))
        extracted.append((b, c))
        bs.append(b)
        subm2 = jnp.where((i1c == lane) & (i0c == c), neg_inf, subm)
        newrow = jnp.max(subm2, axis=0, keepdims=True)
        cm1 = jnp.where(i0r == chunk, newrow, cm1)

    firsts = []
    for j in range(_K2):
        fj = None
        for k in range(j):
            ne = bs[j] != bs[k]
            fj = ne if fj is None else (fj & ne)
        firsts.append(jnp.bool_(True) if fj is None else fj)
    ranks = []
    acc = jnp.int32(0)
    for j in range(_K2):
        ranks.append(acc)
        acc = acc + firsts[j].astype(jnp.int32)

    i1b = jax.lax.broadcasted_iota(jnp.int32, (4, 128), 1)
    for s in range(_NP):
        sel = jnp.int32(0)
        val = jnp.float32(0.0)
        for j in range(_K2):
            hit = firsts[j] & (ranks[j] == s)
            sel = jnp.where(hit, bs[j], sel)
            val = jnp.where(hit, jnp.float32(1.0), val)
        sc = sel // 128
        sl = sel - sc * 128
        bchunk = bx_ref[:, pl.ds(pl.multiple_of(sc * 128, 128), 128)]
        picked = jnp.where(i1b == sl, bchunk, neg_inf)
        for k in range(4):
            ck = jnp.max(picked[k:k + 1, :])
            sel_ref[pl.ds(s, 1), k:k + 1] = jnp.full((1, 1), ck, jnp.float32)
        sel_ref[pl.ds(s, 1), 4:5] = jnp.full((1, 1), val, jnp.float32)


def _interp_matrix(c1, c2, size, cell_idx):
    pos = jax.lax.broadcasted_iota(jnp.int32, (56, size), 1)
    t = cell_idx.astype(jnp.float32) / jnp.float32(_POOL - 1)
    a = c1 * jnp.float32(size - 1)
    span = (c2 - c1) * jnp.float32(size - 1)
    in_c = a + t * span
    c0f = jnp.floor(in_c)
    w = in_c - c0f
    c0 = jnp.clip(c0f, 0, size - 1).astype(jnp.int32)
    c1i = jnp.clip(c0f + 1.0, 0, size - 1).astype(jnp.int32)
    return ((1.0 - w) * (pos == c0).astype(jnp.float32)
            + w * (pos == c1i).astype(jnp.float32))


def _fused_kernel(s_ref, bx_ref, f3_ref, f4_ref, f5_ref, f6_ref, out_ref,
                  cm1_ref, sel_ref):
    g = pl.program_id(0)

    @pl.when(g == 0)
    def _():
        _select(s_ref, bx_ref, cm1_ref, sel_ref)

    for sub_b in range(2):
        b = g * 2 + sub_b
        x1 = sel_ref[pl.ds(b, 1), 0:1]
        y1 = sel_ref[pl.ds(b, 1), 1:2]
        x2 = sel_ref[pl.ds(b, 1), 2:3]
        y2 = sel_ref[pl.ds(b, 1), 3:4]
        valid = sel_ref[pl.ds(b, 1), 4:5]

        acc = None
        for f_ref, size in ((f3_ref, 64), (f4_ref, 32), (f5_ref, 16),
                            (f6_ref, 8)):
            r0 = jax.lax.broadcasted_iota(jnp.int32, (56, size), 0)
            iy = r0 // _POOL
            ix = r0 % _POOL
            wy = _interp_matrix(y1, y2, size, iy) * valid
            wx = _interp_matrix(x1, x2, size, ix)
            k = (wy[:, :, None] * wx[:, None, :]).reshape(56, size * size)
            lvl = jnp.dot(k, f_ref[:, :])
            acc = lvl if acc is None else jnp.maximum(acc, lvl)
        for i in range(_POOL):
            out_ref[0, sub_b, i, :, :] = acc[i * _POOL:(i + 1) * _POOL, :]


def kernel(feature_p3, feature_p4, feature_p5, feature_p6, boxes, nms_classification):
    s_t = nms_classification[0].T
    bx_t = boxes[0].T
    fr = [f[0].reshape(-1, 256) for f in
          (feature_p3, feature_p4, feature_p5, feature_p6)]

    out = pl.pallas_call(
        _fused_kernel,
        grid=(_NP // 2,),
        in_specs=[
            pl.BlockSpec((_NC, _NB), lambda g: (0, 0)),
            pl.BlockSpec((4, _NB), lambda g: (0, 0)),
            pl.BlockSpec((64 * 64, 256), lambda g: (0, 0)),
            pl.BlockSpec((32 * 32, 256), lambda g: (0, 0)),
            pl.BlockSpec((16 * 16, 256), lambda g: (0, 0)),
            pl.BlockSpec((8 * 8, 256), lambda g: (0, 0)),
        ],
        out_specs=pl.BlockSpec((1, 2, _POOL, _POOL, 256),
                               lambda g: (0, g, 0, 0, 0)),
        out_shape=jax.ShapeDtypeStruct((1, _NP, _POOL, _POOL, 256),
                                       jnp.float32),
        scratch_shapes=[
            pltpu.VMEM((_NROW, 128), jnp.float32),
            pltpu.VMEM((16, 128), jnp.float32),
        ],
    )(s_t, bx_t, fr[0], fr[1], fr[2], fr[3])

    return out


# speedup vs baseline: 58.7405x; 1.0424x over previous
import jax
import jax.numpy as jnp
from jax.experimental import pallas as pl
from jax.experimental.pallas import tpu as pltpu

_POOL = 7
_K2 = 20
_NP = 10
_THR = 0.05
_NC = 80
_NB = 20000
_NCH = 156
_TAIL = _NB - _NCH * 128
_NROW = 160


def _select(s_ref, bx_ref, cm1_ref, sel_ref):
    neg_inf = jnp.float32(-jnp.inf)
    big = jnp.int32(1 << 30)

    i0c = jax.lax.broadcasted_iota(jnp.int32, (_NC, 128), 0)
    i1c = jax.lax.broadcasted_iota(jnp.int32, (_NC, 128), 1)

    def body(i, carry):
        blk = s_ref[:, pl.ds(pl.multiple_of(i * 512, 512), 512)]
        cmx = jnp.max(blk, axis=0, keepdims=True)
        cmx = jnp.where(cmx > _THR, cmx, neg_inf)
        for k in range(4):
            cm1_ref[pl.ds(i * 4 + k, 1), :] = cmx[:, k * 128:(k + 1) * 128]
        return carry

    jax.lax.fori_loop(0, 38, body, 0)
    cm1_ref[pl.ds(152, 8), :] = jnp.full((8, 128), neg_inf, jnp.float32)
    tb = s_ref[:, pl.ds(_NCH * 128, _TAIL)]
    tmax = jnp.max(tb, axis=0, keepdims=True)
    tmax = jnp.where(tmax > _THR, tmax, neg_inf)
    cm1_ref[pl.ds(_NCH, 1), :] = jnp.concatenate(
        [tmax, jnp.full((1, 128 - _TAIL), neg_inf, jnp.float32)], axis=1)
    for i in range(152, _NCH):
        blk = s_ref[:, i * 128:(i + 1) * 128]
        cmx = jnp.max(blk, axis=0, keepdims=True)
        cm1_ref[pl.ds(i, 1), :] = jnp.where(cmx > _THR, cmx, neg_inf)

    cm1 = cm1_ref[:, :]
    i0r = jax.lax.broadcasted_iota(jnp.int32, (_NROW, 128), 0)
    i1r = jax.lax.broadcasted_iota(jnp.int32, (_NROW, 128), 1)
    boxid = i0r * 128 + i1r

    extracted = []
    bs = []
    for _ in range(_K2):
        m = jnp.max(cm1)
        b = jnp.min(jnp.where(cm1 == m, boxid, big))
        chunk = b // 128
        lane = b - chunk * 128
        sub = s_ref[:, pl.ds(pl.multiple_of(chunk * 128, 128), 128)]
        subm = jnp.where((sub > _THR) & (chunk * 128 + i1c < _NB), sub,
                         neg_inf)
        for (bk, ck) in extracted:
            subm = jnp.where((chunk * 128 + i1c == bk) & (i0c == ck),
                             neg_inf, subm)
        c = jnp.min(jnp.where((i1c == lane) & (subm == m), i0c, big))
        extracted.append((b, c))
        bs.append(b)
        subm2 = jnp.where((i1c == lane) & (i0c == c), neg_inf, subm)
        newrow = jnp.max(subm2, axis=0, keepdims=True)
        cm1 = jnp.where(i0r == chunk, newrow, cm1)

    firsts = []
    for j in range(_K2):
        fj = None
        for k in range(j):
            ne = bs[j] != bs[k]
            fj = ne if fj is None else (fj & ne)
        firsts.append(jnp.bool_(True) if fj is None else fj)
    ranks = []
    acc = jnp.int32(0)
    for j in range(_K2):
        ranks.append(acc)
        acc = acc + firsts[j].astype(jnp.int32)

    i1b = jax.lax.broadcasted_iota(jnp.int32, (4, 128), 1)
    for s in range(_NP):
        sel = jnp.int32(0)
        val = jnp.float32(0.0)
        for j in range(_K2):
            hit = firsts[j] & (ranks[j] == s)
            sel = jnp.where(hit, bs[j], sel)
            val = jnp.where(hit, jnp.float32(1.0), val)
        sc = sel // 128
        sl = sel - sc * 128
        bchunk = bx_ref[:, pl.ds(pl.multiple_of(sc * 128, 128), 128)]
        picked = jnp.where(i1b == sl, bchunk, neg_inf)
        for k in range(4):
            ck = jnp.max(picked[k:k + 1, :])
            sel_ref[pl.ds(s, 1), k:k + 1] = jnp.full((1, 1), ck, jnp.float32)
        sel_ref[pl.ds(s, 1), 4:5] = jnp.full((1, 1), val, jnp.float32)


def _interp_matrix(c1, c2, size, cell_idx):
    pos = jax.lax.broadcasted_iota(jnp.int32, (56, size), 1)
    t = cell_idx.astype(jnp.float32) / jnp.float32(_POOL - 1)
    a = c1 * jnp.float32(size - 1)
    span = (c2 - c1) * jnp.float32(size - 1)
    in_c = a + t * span
    c0f = jnp.floor(in_c)
    w = in_c - c0f
    c0 = jnp.clip(c0f, 0, size - 1).astype(jnp.int32)
    c1i = jnp.clip(c0f + 1.0, 0, size - 1).astype(jnp.int32)
    return ((1.0 - w) * (pos == c0).astype(jnp.float32)
            + w * (pos == c1i).astype(jnp.float32))


def _fused_kernel(s_ref, bx_ref, f3_ref, f4_ref, f5_ref, f6_ref, out_ref,
                  cm1_ref, sel_ref):
    g = pl.program_id(0)

    @pl.when(g == 0)
    def _():
        _select(s_ref, bx_ref, cm1_ref, sel_ref)

    for sub_b in range(2):
        b = g * 2 + sub_b
        x1 = sel_ref[pl.ds(b, 1), 0:1]
        y1 = sel_ref[pl.ds(b, 1), 1:2]
        x2 = sel_ref[pl.ds(b, 1), 2:3]
        y2 = sel_ref[pl.ds(b, 1), 3:4]
        valid = sel_ref[pl.ds(b, 1), 4:5]

        acc = None
        for f_ref, size in ((f3_ref, 64), (f4_ref, 32), (f5_ref, 16),
                            (f6_ref, 8)):
            r0 = jax.lax.broadcasted_iota(jnp.int32, (56, size), 0)
            iy = r0 // _POOL
            ix = r0 % _POOL
            wy = _interp_matrix(y1, y2, size, iy) * valid
            wx = _interp_matrix(x1, x2, size, ix)
            k = (wy[:, :, None] * wx[:, None, :]).reshape(56, size * size)
            lvl = jnp.dot(k, f_ref[:, :])
            acc = lvl if acc is None else jnp.maximum(acc, lvl)
        for i in range(_POOL):
            out_ref[0, sub_b, i, :, :] = acc[i * _POOL:(i + 1) * _POOL, :]


def kernel(feature_p3, feature_p4, feature_p5, feature_p6, boxes, nms_classification):
    s_t = nms_classification[0].T
    bx_t = boxes[0].T
    fr = [f[0].reshape(-1, 256) for f in
          (feature_p3, feature_p4, feature_p5, feature_p6)]

    out = pl.pallas_call(
        _fused_kernel,
        grid=(_NP // 2,),
        in_specs=[
            pl.BlockSpec((_NC, _NB), lambda g: (0, 0)),
            pl.BlockSpec((4, _NB), lambda g: (0, 0)),
            pl.BlockSpec((64 * 64, 256), lambda g: (0, 0)),
            pl.BlockSpec((32 * 32, 256), lambda g: (0, 0)),
            pl.BlockSpec((16 * 16, 256), lambda g: (0, 0)),
            pl.BlockSpec((8 * 8, 256), lambda g: (0, 0)),
        ],
        out_specs=pl.BlockSpec((1, 2, _POOL, _POOL, 256),
                               lambda g: (0, g, 0, 0, 0)),
        out_shape=jax.ShapeDtypeStruct((1, _NP, _POOL, _POOL, 256),
                                       jnp.float32),
        scratch_shapes=[
            pltpu.VMEM((_NROW, 128), jnp.float32),
            pltpu.VMEM((16, 128), jnp.float32),
        ],
    )(s_t, bx_t, fr[0], fr[1], fr[2], fr[3])

    return out


# speedup vs baseline: 58.7523x; 1.0002x over previous
import jax
import jax.numpy as jnp
from jax.experimental import pallas as pl
from jax.experimental.pallas import tpu as pltpu

_POOL = 7
_K2 = 20
_NP = 10
_THR = 0.05
_NC = 80
_NB = 20000
_NCH = 156
_TAIL = _NB - _NCH * 128
_NROW = 160


def _select(s_ref, bx_ref, cm1_ref, sel_ref):
    neg_inf = jnp.float32(-jnp.inf)
    big = jnp.int32(1 << 30)

    i0c = jax.lax.broadcasted_iota(jnp.int32, (_NC, 128), 0)
    i1c = jax.lax.broadcasted_iota(jnp.int32, (_NC, 128), 1)

    def body(i, carry):
        blk = s_ref[:, pl.ds(pl.multiple_of(i * 512, 512), 512)]
        cmx = jnp.max(blk, axis=0, keepdims=True)
        cmx = jnp.where(cmx > _THR, cmx, neg_inf)
        for k in range(4):
            cm1_ref[pl.ds(i * 4 + k, 1), :] = cmx[:, k * 128:(k + 1) * 128]
        return carry

    jax.lax.fori_loop(0, 38, body, 0)
    cm1_ref[pl.ds(152, 8), :] = jnp.full((8, 128), neg_inf, jnp.float32)
    tb = s_ref[:, pl.ds(_NCH * 128, _TAIL)]
    tmax = jnp.max(tb, axis=0, keepdims=True)
    tmax = jnp.where(tmax > _THR, tmax, neg_inf)
    cm1_ref[pl.ds(_NCH, 1), :] = jnp.concatenate(
        [tmax, jnp.full((1, 128 - _TAIL), neg_inf, jnp.float32)], axis=1)
    for i in range(152, _NCH):
        blk = s_ref[:, i * 128:(i + 1) * 128]
        cmx = jnp.max(blk, axis=0, keepdims=True)
        cm1_ref[pl.ds(i, 1), :] = jnp.where(cmx > _THR, cmx, neg_inf)

    cm1 = cm1_ref[:, :]
    i0r = jax.lax.broadcasted_iota(jnp.int32, (_NROW, 128), 0)
    i1r = jax.lax.broadcasted_iota(jnp.int32, (_NROW, 128), 1)
    boxid = i0r * 128 + i1r

    extracted = []
    bs = []
    for _ in range(_K2):
        m = jnp.max(cm1)
        b = jnp.min(jnp.where(cm1 == m, boxid, big))
        chunk = b // 128
        lane = b - chunk * 128
        sub = s_ref[:, pl.ds(pl.multiple_of(chunk * 128, 128), 128)]
        subm = jnp.where((sub > _THR) & (chunk * 128 + i1c < _NB), sub,
                         neg_inf)
        for (bk, ck) in extracted:
            subm = jnp.where((chunk * 128 + i1c == bk) & (i0c == ck),
                             neg_inf, subm)
        c = jnp.min(jnp.where((i1c == lane) & (subm == m), i0c, big))
        extracted.append((b, c))
        bs.append(b)
        subm2 = jnp.where((i1c == lane) & (i0c == c), neg_inf, subm)
        newrow = jnp.max(subm2, axis=0, keepdims=True)
        cm1 = jnp.where(i0r == chunk, newrow, cm1)

    firsts = []
    for j in range(_K2):
        fj = None
        for k in range(j):
            ne = bs[j] != bs[k]
            fj = ne if fj is None else (fj & ne)
        firsts.append(jnp.bool_(True) if fj is None else fj)
    ranks = []
    acc = jnp.int32(0)
    for j in range(_K2):
        ranks.append(acc)
        acc = acc + firsts[j].astype(jnp.int32)

    i1b = jax.lax.broadcasted_iota(jnp.int32, (4, 128), 1)
    for s in range(_NP):
        sel = jnp.int32(0)
        val = jnp.float32(0.0)
        for j in range(_K2):
            hit = firsts[j] & (ranks[j] == s)
            sel = jnp.where(hit, bs[j], sel)
            val = jnp.where(hit, jnp.float32(1.0), val)
        sc = sel // 128
        sl = sel - sc * 128
        bchunk = bx_ref[:, pl.ds(pl.multiple_of(sc * 128, 128), 128)]
        picked = jnp.where(i1b == sl, bchunk, neg_inf)
        for k in range(4):
            ck = jnp.max(picked[k:k + 1, :])
            sel_ref[pl.ds(s, 1), k:k + 1] = jnp.full((1, 1), ck, jnp.float32)
        sel_ref[pl.ds(s, 1), 4:5] = jnp.full((1, 1), val, jnp.float32)


def _interp_matrix(c1, c2, size, cell_idx):
    pos = jax.lax.broadcasted_iota(jnp.int32, (56, size), 1)
    t = cell_idx.astype(jnp.float32) / jnp.float32(_POOL - 1)
    a = c1 * jnp.float32(size - 1)
    span = (c2 - c1) * jnp.float32(size - 1)
    in_c = a + t * span
    c0f = jnp.floor(in_c)
    w = in_c - c0f
    c0 = jnp.clip(c0f, 0, size - 1).astype(jnp.int32)
    c1i = jnp.clip(c0f + 1.0, 0, size - 1).astype(jnp.int32)
    return ((1.0 - w) * (pos == c0).astype(jnp.float32)
            + w * (pos == c1i).astype(jnp.float32))


def _fused_kernel(s_ref, bx_ref, f3_ref, f4_ref, f5_ref, f6_ref, out_ref,
                  cm1_ref, sel_ref):
    g = pl.program_id(0)

    @pl.when(g == 0)
    def _():
        _select(s_ref, bx_ref, cm1_ref, sel_ref)

    for sub_b in range(5):
        b = g * 5 + sub_b
        x1 = sel_ref[pl.ds(b, 1), 0:1]
        y1 = sel_ref[pl.ds(b, 1), 1:2]
        x2 = sel_ref[pl.ds(b, 1), 2:3]
        y2 = sel_ref[pl.ds(b, 1), 3:4]
        valid = sel_ref[pl.ds(b, 1), 4:5]

        acc = None
        for f_ref, size in ((f3_ref, 64), (f4_ref, 32), (f5_ref, 16),
                            (f6_ref, 8)):
            r0 = jax.lax.broadcasted_iota(jnp.int32, (56, size), 0)
            iy = r0 // _POOL
            ix = r0 % _POOL
            wy = _interp_matrix(y1, y2, size, iy) * valid
            wx = _interp_matrix(x1, x2, size, ix)
            k = (wy[:, :, None] * wx[:, None, :]).reshape(56, size * size)
            lvl = jnp.dot(k, f_ref[:, :])
            acc = lvl if acc is None else jnp.maximum(acc, lvl)
        for i in range(_POOL):
            out_ref[0, sub_b, i, :, :] = acc[i * _POOL:(i + 1) * _POOL, :]


def kernel(feature_p3, feature_p4, feature_p5, feature_p6, boxes, nms_classification):
    s_t = nms_classification[0].T
    bx_t = boxes[0].T
    fr = [f[0].reshape(-1, 256) for f in
          (feature_p3, feature_p4, feature_p5, feature_p6)]

    out = pl.pallas_call(
        _fused_kernel,
        grid=(_NP // 5,),
        in_specs=[
            pl.BlockSpec((_NC, _NB), lambda g: (0, 0)),
            pl.BlockSpec((4, _NB), lambda g: (0, 0)),
            pl.BlockSpec((64 * 64, 256), lambda g: (0, 0)),
            pl.BlockSpec((32 * 32, 256), lambda g: (0, 0)),
            pl.BlockSpec((16 * 16, 256), lambda g: (0, 0)),
            pl.BlockSpec((8 * 8, 256), lambda g: (0, 0)),
        ],
        out_specs=pl.BlockSpec((1, 5, _POOL, _POOL, 256),
                               lambda g: (0, g, 0, 0, 0)),
        out_shape=jax.ShapeDtypeStruct((1, _NP, _POOL, _POOL, 256),
                                       jnp.float32),
        scratch_shapes=[
            pltpu.VMEM((_NROW, 128), jnp.float32),
            pltpu.VMEM((16, 128), jnp.float32),
        ],
    )(s_t, bx_t, fr[0], fr[1], fr[2], fr[3])

    return out
